# baseline (speedup 1.0000x reference)
import functools

import jax
import jax.numpy as jnp
from jax import lax
from jax.experimental import pallas as pl
from jax.experimental.pallas import tpu as pltpu
from jax.experimental.pallas import tpu_sc as plsc

N = 10000
E = 320000
D = 128
NC = 2
NS = 16
NW = NC * NS
EPW = E // NW
CH = 80
NCHUNK = EPW // CH
K = 5
NBLK = NCHUNK // K
B = 4
NIDX = 3
NPAD = 10112
RPS = NPAD // NS
DPAD = 10240
DRPS = DPAD // NS

_mesh = plsc.VectorSubcoreMesh(core_axis_name="c", subcore_axis_name="s")


def _deg_body(eidx_hbm, zvec_hbm, parts_hbm, idx_v, ones_v, acc_sh, ss):
    c = lax.axis_index("c")
    s = lax.axis_index("s")
    wid = s * NC + c
    pltpu.sync_copy(zvec_hbm, acc_sh.at[pl.ds(s * DRPS, DRPS)])
    pltpu.sync_copy(eidx_hbm.at[wid], idx_v)
    one16 = jnp.ones((16,), jnp.float32)
    for j in range(8):
        ones_v[pl.ds(j * 16, 16)] = one16
    plsc.subcore_barrier()

    def body(k, carry):
        m = k // K
        kk = lax.rem(k, K)
        pltpu.async_copy(ones_v.at[pl.ds(0, CH)],
                         acc_sh.at[idx_v.at[m].at[1].at[kk]], ss, add=True)
        return carry

    lax.fori_loop(0, NCHUNK, body, 0)

    def dbody(k, carry):
        pltpu.make_async_copy(zvec_hbm.at[pl.ds(0, CH)], ones_v.at[pl.ds(0, CH)],
                              ss).wait()
        return carry

    lax.fori_loop(0, NCHUNK, dbody, 0)
    plsc.subcore_barrier()
    pltpu.sync_copy(acc_sh.at[pl.ds(s * DRPS, DRPS)],
                    parts_hbm.at[c].at[pl.ds(s * DRPS, DRPS)])


_deg_call = functools.partial(
    pl.kernel,
    out_type=jax.ShapeDtypeStruct((NC, DPAD), jnp.float32),
    mesh=_mesh,
    scratch_types=[
        pltpu.VMEM((NBLK, 2, K, CH), jnp.int32),
        pltpu.VMEM((128,), jnp.float32),
        pltpu.VMEM_SHARED((DPAD,), jnp.float32),
        pltpu.SemaphoreType.DMA,
    ],
)(_deg_body)


def _hop_body(g_hbm, eidx_hbm, zrows_hbm, out_hbm,
              idx_v, bufs_v, acc_sh, sg, ss, si):
    c = lax.axis_index("c")
    s = lax.axis_index("s")
    wid = s * NC + c
    zdesc = pltpu.async_copy(zrows_hbm, acc_sh.at[pl.ds(s * RPS, RPS)], ss)

    def drain_gather():
        pltpu.make_async_copy(g_hbm.at[pl.ds(0, CH)], bufs_v.at[0],
                              sg).wait()

    def drain_scatter():
        pltpu.make_async_copy(g_hbm.at[pl.ds(0, CH)], bufs_v.at[0],
                              ss).wait()

    def drain_idx():
        pltpu.make_async_copy(eidx_hbm.at[0].at[0], idx_v.at[0], si).wait()

    pltpu.sync_copy(eidx_hbm.at[wid].at[0], idx_v.at[0])
    pltpu.async_copy(eidx_hbm.at[wid].at[1], idx_v.at[1], si)
    pltpu.async_copy(g_hbm.at[idx_v.at[0].at[0].at[0]], bufs_v.at[0], sg)
    pltpu.async_copy(g_hbm.at[idx_v.at[0].at[0].at[1]], bufs_v.at[1], sg)
    zdesc.wait()
    plsc.subcore_barrier()

    def body(k, carry):
        m = k // K
        kk = lax.rem(k, K)
        slot = lax.rem(m, NIDX)
        b = lax.rem(k, B)
        drain_gather()

        @pl.when((kk == 2) & (m + 2 < NBLK))
        def _():
            pltpu.async_copy(eidx_hbm.at[wid].at[m + 2],
                             idx_v.at[lax.rem(m + 2, NIDX)], si)

        pltpu.async_copy(bufs_v.at[b],
                         acc_sh.at[idx_v.at[slot].at[1].at[kk]],
                         ss, add=True)

        @pl.when(k >= 2)
        def _():
            drain_scatter()

        k2 = k + 2
        m2 = k2 // K
        kk2 = lax.rem(k2, K)

        @pl.when((k2 < NCHUNK) & (kk2 == 0))
        def _():
            drain_idx()

        @pl.when(k2 < NCHUNK)
        def _():
            pltpu.async_copy(
                g_hbm.at[idx_v.at[lax.rem(m2, NIDX)].at[0].at[kk2]],
                bufs_v.at[lax.rem(k2, B)], sg)

        return carry

    lax.fori_loop(0, NCHUNK, body, 0)
    drain_scatter()
    drain_scatter()
    plsc.subcore_barrier()
    pltpu.sync_copy(acc_sh.at[pl.ds(s * RPS, RPS)],
                    out_hbm.at[c].at[pl.ds(s * RPS, RPS)])


_hop_call = functools.partial(
    pl.kernel,
    out_type=jax.ShapeDtypeStruct((NC, NPAD, D), jnp.float32),
    mesh=_mesh,
    scratch_types=[
        pltpu.VMEM((NIDX, 2, K, CH), jnp.int32),
        pltpu.VMEM((B, CH, D), jnp.float32),
        pltpu.VMEM_SHARED((NPAD, D), jnp.float32),
        pltpu.SemaphoreType.DMA,
        pltpu.SemaphoreType.DMA,
        pltpu.SemaphoreType.DMA,
    ],
)(_hop_body)


def _mm_body(x_ref, w_ref, xw_ref):
    xw_ref[...] = lax.dot_general(
        x_ref[...], w_ref[...], (((1,), (1,)), ((), ())),
        preferred_element_type=jnp.float32)


def _prep_body(parts_ref, xw_ref, g0_ref, dinv_ref):
    deg = parts_ref[0, :N] + parts_ref[1, :N] + 1.0
    dinv = lax.rsqrt(deg)[:, None]
    dinv_ref[...] = dinv
    g0_ref[...] = xw_ref[...] * dinv


def _mid_body(p_ref, g0_ref, dinv_ref, g1_ref):
    s = dinv_ref[...]
    u1 = p_ref[0] + p_ref[1] + g0_ref[...]
    g1_ref[...] = u1 * (s * s)


def _out_body(p_ref, g1_ref, dinv_ref, b_ref, o_ref):
    u2 = p_ref[0] + p_ref[1] + g1_ref[...]
    o_ref[...] = u2 * dinv_ref[...] + b_ref[...]


@jax.jit
def _run(x, src, dst, W, b):
    src3 = src.reshape(NW, NBLK, 1, K, CH)
    dst3 = dst.reshape(NW, NBLK, 1, K, CH)
    eidx = jnp.concatenate([src3, dst3], axis=2)
    zrows = jnp.zeros((RPS, D), jnp.float32)
    zvec = jnp.zeros((DRPS,), jnp.float32)
    b2 = b.reshape(1, D)

    RB = 1000
    NG = N // RB

    xw = pl.pallas_call(
        _mm_body,
        grid=(NG,),
        in_specs=[pl.BlockSpec((RB, D), lambda i: (i, 0)),
                  pl.BlockSpec((D, D), lambda i: (0, 0))],
        out_specs=pl.BlockSpec((RB, D), lambda i: (i, 0)),
        out_shape=jax.ShapeDtypeStruct((N, D), jnp.float32),
    )(x, W)
    parts = _deg_call(eidx, zvec)

    g0, dinv = pl.pallas_call(
        _prep_body,
        out_shape=(jax.ShapeDtypeStruct((N, D), jnp.float32),
                   jax.ShapeDtypeStruct((N, 1), jnp.float32)),
    )(parts, xw)

    p1 = _hop_call(g0, eidx, zrows)
    g1 = pl.pallas_call(
        _mid_body,
        grid=(NG,),
        in_specs=[pl.BlockSpec((NC, RB, D), lambda i: (0, i, 0)),
                  pl.BlockSpec((RB, D), lambda i: (i, 0)),
                  pl.BlockSpec((RB, 1), lambda i: (i, 0))],
        out_specs=pl.BlockSpec((RB, D), lambda i: (i, 0)),
        out_shape=jax.ShapeDtypeStruct((N, D), jnp.float32),
    )(p1, g0, dinv)

    p2 = _hop_call(g1, eidx, zrows)
    out = pl.pallas_call(
        _out_body,
        grid=(NG,),
        in_specs=[pl.BlockSpec((NC, RB, D), lambda i: (0, i, 0)),
                  pl.BlockSpec((RB, D), lambda i: (i, 0)),
                  pl.BlockSpec((RB, 1), lambda i: (i, 0)),
                  pl.BlockSpec((1, D), lambda i: (0, 0))],
        out_specs=pl.BlockSpec((RB, D), lambda i: (i, 0)),
        out_shape=jax.ShapeDtypeStruct((N, D), jnp.float32),
    )(p2, g1, dinv, b2)
    return out


def kernel(x, edge_index, W, b):
    src = edge_index[0].astype(jnp.int32)
    dst = edge_index[1].astype(jnp.int32)
    return _run(x, src, dst, W, b)


# speedup vs baseline: 1.0154x; 1.0154x over previous
import functools

import jax
import jax.numpy as jnp
from jax import lax
from jax.experimental import pallas as pl
from jax.experimental.pallas import tpu as pltpu
from jax.experimental.pallas import tpu_sc as plsc

N = 10000
E = 320000
D = 128
NC = 2
NS = 16
NW = NC * NS
EPW = E // NW
CH = 80
NCHUNK = EPW // CH
K = 5
NBLK = NCHUNK // K
B = 4
NIDX = 3
NPAD = 10112
RPS = NPAD // NS
DPAD = 10240
DRPS = DPAD // NS

_mesh = plsc.VectorSubcoreMesh(core_axis_name="c", subcore_axis_name="s")


def _deg_body(eidx_hbm, zvec_hbm, parts_hbm, idx_v, ones_v, acc_sh, ss):
    c = lax.axis_index("c")
    s = lax.axis_index("s")
    wid = s * NC + c
    pltpu.sync_copy(zvec_hbm, acc_sh.at[pl.ds(s * DRPS, DRPS)])
    pltpu.sync_copy(eidx_hbm.at[wid], idx_v)
    one16 = jnp.ones((16,), jnp.float32)
    for j in range(8):
        ones_v[pl.ds(j * 16, 16)] = one16
    plsc.subcore_barrier()

    def body(k, carry):
        m = k // K
        kk = lax.rem(k, K)
        pltpu.async_copy(ones_v.at[pl.ds(0, CH)],
                         acc_sh.at[idx_v.at[m].at[1].at[kk]], ss, add=True)
        return carry

    lax.fori_loop(0, NCHUNK, body, 0)

    def dbody(k, carry):
        pltpu.make_async_copy(zvec_hbm.at[pl.ds(0, CH)], ones_v.at[pl.ds(0, CH)],
                              ss).wait()
        return carry

    lax.fori_loop(0, NCHUNK, dbody, 0)
    plsc.subcore_barrier()
    pltpu.sync_copy(acc_sh.at[pl.ds(s * DRPS, DRPS)],
                    parts_hbm.at[c].at[pl.ds(s * DRPS, DRPS)])


_deg_call = functools.partial(
    pl.kernel,
    out_type=jax.ShapeDtypeStruct((NC, DPAD), jnp.float32),
    mesh=_mesh,
    scratch_types=[
        pltpu.VMEM((NBLK, 2, K, CH), jnp.int32),
        pltpu.VMEM((128,), jnp.float32),
        pltpu.VMEM_SHARED((DPAD,), jnp.float32),
        pltpu.SemaphoreType.DMA,
    ],
)(_deg_body)


def _hop_body(g_hbm, eidx_hbm, zrows_hbm, out_hbm,
              idx_v, bufs_v, acc_sh, sg, ss, si):
    c = lax.axis_index("c")
    s = lax.axis_index("s")
    wid = s * NC + c
    zdesc = pltpu.async_copy(zrows_hbm, acc_sh.at[pl.ds(s * RPS, RPS)], ss)

    def drain_gather():
        pltpu.make_async_copy(g_hbm.at[pl.ds(0, CH)], bufs_v.at[0],
                              sg).wait()

    def drain_scatter():
        pltpu.make_async_copy(g_hbm.at[pl.ds(0, CH)], bufs_v.at[0],
                              ss).wait()

    def drain_idx():
        pltpu.make_async_copy(eidx_hbm.at[0].at[0], idx_v.at[0], si).wait()

    pltpu.sync_copy(eidx_hbm.at[wid].at[0], idx_v.at[0])
    pltpu.async_copy(eidx_hbm.at[wid].at[1], idx_v.at[1], si)
    pltpu.async_copy(g_hbm.at[idx_v.at[0].at[0].at[0]], bufs_v.at[0], sg)
    pltpu.async_copy(g_hbm.at[idx_v.at[0].at[0].at[1]], bufs_v.at[1], sg)
    zdesc.wait()
    plsc.subcore_barrier()

    def body(k, carry):
        m = k // K
        kk = lax.rem(k, K)
        slot = lax.rem(m, NIDX)
        b = lax.rem(k, B)
        drain_gather()

        @pl.when((kk == 2) & (m + 2 < NBLK))
        def _():
            pltpu.async_copy(eidx_hbm.at[wid].at[m + 2],
                             idx_v.at[lax.rem(m + 2, NIDX)], si)

        pltpu.async_copy(bufs_v.at[b],
                         acc_sh.at[idx_v.at[slot].at[1].at[kk]],
                         ss, add=True)

        @pl.when(k >= 2)
        def _():
            drain_scatter()

        k2 = k + 2
        m2 = k2 // K
        kk2 = lax.rem(k2, K)

        @pl.when((k2 < NCHUNK) & (kk2 == 0))
        def _():
            drain_idx()

        @pl.when(k2 < NCHUNK)
        def _():
            pltpu.async_copy(
                g_hbm.at[idx_v.at[lax.rem(m2, NIDX)].at[0].at[kk2]],
                bufs_v.at[lax.rem(k2, B)], sg)

        return carry

    lax.fori_loop(0, NCHUNK, body, 0)
    drain_scatter()
    drain_scatter()
    plsc.subcore_barrier()
    pltpu.sync_copy(acc_sh.at[pl.ds(s * RPS, RPS)],
                    out_hbm.at[c].at[pl.ds(s * RPS, RPS)])


_hop_call = functools.partial(
    pl.kernel,
    out_type=jax.ShapeDtypeStruct((NC, NPAD, D), jnp.float32),
    mesh=_mesh,
    scratch_types=[
        pltpu.VMEM((NIDX, 2, K, CH), jnp.int32),
        pltpu.VMEM((B, CH, D), jnp.float32),
        pltpu.VMEM_SHARED((NPAD, D), jnp.float32),
        pltpu.SemaphoreType.DMA,
        pltpu.SemaphoreType.DMA,
        pltpu.SemaphoreType.DMA,
    ],
)(_hop_body)


def _mm_body(x_ref, w_ref, xw_ref):
    xw_ref[...] = lax.dot_general(
        x_ref[...], w_ref[...], (((1,), (1,)), ((), ())),
        preferred_element_type=jnp.float32)


def _prep_body(parts_ref, xw_ref, g0_ref, dinv_ref):
    deg = parts_ref[0, :N] + parts_ref[1, :N] + 1.0
    dinv = lax.rsqrt(deg)[:, None]
    dinv_ref[...] = dinv
    g0_ref[...] = xw_ref[...] * dinv


def _mid_body(p_ref, g0_ref, dinv_ref, g1_ref):
    s = dinv_ref[...]
    u1 = p_ref[0, :N] + p_ref[1, :N] + g0_ref[...]
    g1_ref[...] = u1 * (s * s)


def _out_body(p_ref, g1_ref, dinv_ref, b_ref, o_ref):
    u2 = p_ref[0, :N] + p_ref[1, :N] + g1_ref[...]
    o_ref[...] = u2 * dinv_ref[...] + b_ref[...]


@jax.jit
def _run(x, src, dst, W, b):
    src3 = src.reshape(NW, NBLK, 1, K, CH)
    dst3 = dst.reshape(NW, NBLK, 1, K, CH)
    eidx = jnp.concatenate([src3, dst3], axis=2)
    zrows = jnp.zeros((RPS, D), jnp.float32)
    zvec = jnp.zeros((DRPS,), jnp.float32)
    b2 = b.reshape(1, D)

    xw = pl.pallas_call(
        _mm_body,
        out_shape=jax.ShapeDtypeStruct((N, D), jnp.float32),
    )(x, W)
    parts = _deg_call(eidx, zvec)

    g0, dinv = pl.pallas_call(
        _prep_body,
        out_shape=(jax.ShapeDtypeStruct((N, D), jnp.float32),
                   jax.ShapeDtypeStruct((N, 1), jnp.float32)),
    )(parts, xw)

    p1 = _hop_call(g0, eidx, zrows)
    g1 = pl.pallas_call(
        _mid_body,
        out_shape=jax.ShapeDtypeStruct((N, D), jnp.float32),
    )(p1, g0, dinv)

    p2 = _hop_call(g1, eidx, zrows)
    out = pl.pallas_call(
        _out_body,
        out_shape=jax.ShapeDtypeStruct((N, D), jnp.float32),
    )(p2, g1, dinv, b2)
    return out


def kernel(x, edge_index, W, b):
    src = edge_index[0].astype(jnp.int32)
    dst = edge_index[1].astype(jnp.int32)
    return _run(x, src, dst, W, b)
